# baseline (speedup 1.0000x reference)
import functools

import jax
import jax.numpy as jnp
from jax import lax
from jax.experimental import pallas as pl
from jax.experimental.pallas import tpu as pltpu
from jax.experimental.pallas import tpu_sc as plsc

B = 8
N = 500000
NVOX = 4096
PLANE = 4 * NVOX
CH_P = 512
NCH = 61

_MESH = plsc.VectorSubcoreMesh(core_axis_name="c", subcore_axis_name="s")


@functools.partial(
    pl.kernel,
    mesh=_MESH,
    out_type=jax.ShapeDtypeStruct((B * NVOX * 3,), jnp.float32),
    scratch_types=[
        pltpu.VMEM((24, CH_P), jnp.float32),
        pltpu.VMEM((24, CH_P), jnp.float32),
        pltpu.VMEM((24, CH_P), jnp.float32),
        pltpu.VMEM((4 * PLANE,), jnp.float32),
        pltpu.VMEM((NVOX,), jnp.float32),
        pltpu.VMEM((2 * NVOX,), jnp.float32),
        pltpu.VMEM((NVOX * 3,), jnp.float32),
        pltpu.VMEM_SHARED((16, NVOX), jnp.float32),
        pltpu.SemaphoreType.DMA,
        pltpu.SemaphoreType.DMA,
        pltpu.SemaphoreType.DMA,
    ],
    compiler_params=pltpu.CompilerParams(needs_layout_passes=False),
)
def _voxelize(x_hbm, t_hbm, out_hbm, buf0, buf1, buf2, acc, comb, fin2,
              outb, shared, sem0, sem1, sem2):
    core = lax.axis_index("c")
    s = lax.axis_index("s")

    iota = lax.iota(jnp.int32, 16)
    iota3 = iota * 3
    onef = jnp.full((16,), 1.0, jnp.float32)
    z16 = jnp.zeros((16,), jnp.float32)

    pstart0 = (s * 244 + jnp.minimum(s, 2)) * 128

    def _src(ch):
        return x_hbm.at[pl.ds(0, 24), pl.ds(pstart0 + ch * CH_P, CH_P)]

    bufs = (buf0, buf1, buf2)
    sems = (sem0, sem1, sem2)

    pltpu.async_copy(_src(0), bufs[0], sems[0])
    pltpu.async_copy(_src(1), bufs[1], sems[1])

    def _zero(i, carry):
        acc[pl.ds(i * 16, 16)] = z16
        return carry

    lax.fori_loop(0, 4 * PLANE // 16, _zero, 0)

    def _make_group(buf, unroll):
        def _group(g, carry):
            for u in range(unroll):
                sl = pl.ds((g * unroll + u) * 16, 16)
                for bl in range(4):
                    row = core * 4 + bl
                    xv = buf[row, sl]
                    yv = buf[8 + row, sl]
                    zv = buf[16 + row, sl]
                    ix = (xv * 16.0).astype(jnp.int32)
                    iy = (yv * 16.0).astype(jnp.int32)
                    iz = (zv * 16.0).astype(jnp.int32)
                    vid = ((ix << 8) | (iy << 4) | iz) + bl * PLANE
                    plsc.addupdate_scatter(acc, [vid], xv)
                    plsc.addupdate_scatter(acc, [vid + NVOX], yv)
                    plsc.addupdate_scatter(acc, [vid + 2 * NVOX], zv)
                    plsc.addupdate_scatter(acc, [vid + 3 * NVOX], onef)
            return carry

        return _group

    _groups = tuple(_make_group(bf, 2) for bf in bufs)
    _group01 = _make_group(buf0, 1)

    def _tri(i, carry):
        for k in range(3):
            ch = 3 * i + k
            k2 = (k + 2) % 3
            pltpu.make_async_copy(_src(ch), bufs[k], sems[k]).wait()

            @pl.when(ch + 2 < NCH)
            def _(ch=ch, k2=k2):
                pltpu.async_copy(_src(ch + 2), bufs[k2], sems[k2])

            lax.fori_loop(0, CH_P // 32, _groups[k], 0)
        return carry

    lax.fori_loop(0, NCH // 3, _tri, 0)

    pltpu.make_async_copy(_src(NCH - 1), bufs[0], sems[0]).wait()
    lax.fori_loop(0, CH_P // 32, _groups[0], 0)

    @pl.when(s < 2)
    def _():
        pltpu.sync_copy(
            x_hbm.at[pl.ds(0, 24), pl.ds(pstart0 + NCH * CH_P, 128)],
            buf0.at[:, pl.ds(0, 128)])
        lax.fori_loop(0, 8, _group01, 0)

    @pl.when(s == 15)
    def _():
        pltpu.sync_copy(t_hbm, buf0.at[:, pl.ds(0, 128)])
        lax.fori_loop(0, 2, _group01, 0)

    myoff = s * NVOX

    def _mround(i, carry):
        t = lax.rem(s + 1 + i, 16)
        pltpu.sync_copy(acc.at[pl.ds(t * NVOX, NVOX)], shared.at[s])
        plsc.subcore_barrier()
        u = lax.rem(s + 15 - i, 16)
        pltpu.sync_copy(shared.at[u], comb)
        plsc.subcore_barrier()

        def _add(k, carry2):
            sl = pl.ds(myoff + k * 16, 16)
            acc[sl] = acc[sl] + comb[pl.ds(k * 16, 16)]
            return carry2

        lax.fori_loop(0, NVOX // 16, _add, 0)
        return carry

    lax.fori_loop(0, 15, _mround, 0)

    pltpu.sync_copy(acc.at[pl.ds(myoff, NVOX)], shared.at[s])
    plsc.subcore_barrier()

    @pl.when((s & 3) == 0)
    def _():
        bl = s >> 2
        b = core * 4 + bl
        pltpu.sync_copy(shared.at[4 * bl + 1], comb)
        pltpu.sync_copy(shared.at[4 * bl + 2], fin2.at[pl.ds(0, NVOX)])
        pltpu.sync_copy(shared.at[4 * bl + 3], fin2.at[pl.ds(NVOX, NVOX)])

        def _fin(g, carry):
            vb = g * 16
            sx = acc[pl.ds(myoff + vb, 16)]
            sy = comb[pl.ds(vb, 16)]
            sz = fin2[pl.ds(vb, 16)]
            cn = fin2[pl.ds(NVOX + vb, 16)]
            d = jnp.maximum(cn, 1.0)
            m = cn > 1.0
            vout = g * 48 + iota3
            plsc.store_scatter(outb, [vout], jnp.where(m, sx / d, z16))
            plsc.store_scatter(outb, [vout + 1], jnp.where(m, sy / d, z16))
            plsc.store_scatter(outb, [vout + 2], jnp.where(m, sz / d, z16))
            return carry

        lax.fori_loop(0, NVOX // 16, _fin, 0)
        pltpu.sync_copy(outb, out_hbm.at[pl.ds(b * (NVOX * 3), NVOX * 3)])


def kernel(x):
    planes = jnp.transpose(x, (2, 0, 1)).reshape(24, N)
    tail = jnp.pad(planes[:, 3906 * 128:], ((0, 0), (0, 96)))
    out = _voxelize(planes, tail)
    return out.reshape(B, NVOX, 3)
